# Initial kernel scaffold; baseline (speedup 1.0000x reference)
import jax
import jax.numpy as jnp
from jax.experimental import pallas as pl


def kernel(H, sm_rows, sm_cols, sm_vals, sp_rows, sp_cols, sp_vals, W_enc0, b_enc0, W_enc1, b_enc1, W_enc2, b_enc2, W_dec0, b_dec0, W_dec1, b_dec1, W_dec2, b_dec2):
    raise NotImplementedError("write your pallas kernel here")


# speedup vs baseline: 6.7495x; 6.7495x over previous
import jax
import jax.numpy as jnp
from jax.experimental import pallas as pl

_N = 17
_B = 256
_R = _N * _B


def _body(x_ref, a_sm_ref, a_sp_ref,
          w0_ref, b0_ref, w1_ref, b1_ref, w2_ref, b2_ref,
          w3_ref, b3_ref, w4_ref, b4_ref, w5_ref, b5_ref,
          out_ref):
    f32 = jnp.float32

    ii = jax.lax.broadcasted_iota(jnp.int32, (_N, _N), 0)
    jj = jax.lax.broadcasted_iota(jnp.int32, (_N, _N), 1)
    rr = jax.lax.broadcasted_iota(jnp.int32, (_R, _N), 0)
    nn = jax.lax.broadcasted_iota(jnp.int32, (_R, _N), 1)
    tile = (jnp.remainder(rr, _N) == nn).astype(f32)

    def coeff_cols(a):
        lo = jnp.sum(jnp.where(jj == ii - 1, a, 0.0), axis=1, keepdims=True)
        di = jnp.sum(jnp.where(jj == ii, a, 0.0), axis=1, keepdims=True)
        up = jnp.sum(jnp.where(jj == ii + 1, a, 0.0), axis=1, keepdims=True)
        c = jnp.dot(tile, jnp.concatenate([lo, di, up], axis=1),
                    preferred_element_type=f32,
                    precision=jax.lax.Precision.HIGHEST)
        return c[:, 0:1], c[:, 1:2], c[:, 2:3]

    sm_lo, sm_di, sm_up = coeff_cols(a_sm_ref[...])
    sp_lo, sp_di, sp_up = coeff_cols(a_sp_ref[...])

    def layer(x, w_ref, b_ref, lo, di, up):
        y = jnp.dot(x.astype(jnp.bfloat16), w_ref[...].astype(jnp.bfloat16),
                    preferred_element_type=f32) + b_ref[...]
        d = y.shape[1]
        y_prev = jnp.concatenate([jnp.zeros((1, d), f32), y[:-1, :]], axis=0)
        y_next = jnp.concatenate([y[1:, :], jnp.zeros((1, d), f32)], axis=0)
        z = di * y + lo * y_prev + up * y_next
        return jnp.maximum(z, 0.0)

    x = x_ref[...]
    x = layer(x, w0_ref, b0_ref, sm_lo, sm_di, sm_up)
    x = layer(x, w1_ref, b1_ref, sm_lo, sm_di, sm_up)
    x = layer(x, w2_ref, b2_ref, sm_lo, sm_di, sm_up)
    x = layer(x, w3_ref, b3_ref, sp_lo, sp_di, sp_up)
    x = layer(x, w4_ref, b4_ref, sp_lo, sp_di, sp_up)
    x = layer(x, w5_ref, b5_ref, sp_lo, sp_di, sp_up)
    out_ref[...] = x


def kernel(H, sm_rows, sm_cols, sm_vals, sp_rows, sp_cols, sp_vals,
           W_enc0, b_enc0, W_enc1, b_enc1, W_enc2, b_enc2,
           W_dec0, b_dec0, W_dec1, b_dec1, W_dec2, b_dec2):
    f32 = jnp.float32
    a_sm = jnp.zeros((_N, _N), f32).at[sm_rows, sm_cols].add(sm_vals)
    a_sp = jnp.zeros((_N, _N), f32).at[sp_rows, sp_cols].add(sp_vals)

    x = H.reshape(_R, 2)
    wb = (W_enc0, b_enc0.reshape(1, -1), W_enc1, b_enc1.reshape(1, -1),
          W_enc2, b_enc2.reshape(1, -1), W_dec0, b_dec0.reshape(1, -1),
          W_dec1, b_dec1.reshape(1, -1), W_dec2, b_dec2.reshape(1, -1))

    out = pl.pallas_call(
        _body,
        out_shape=jax.ShapeDtypeStruct((_R, 2), f32),
    )(x, a_sm, a_sp, *wb)
    return out.reshape(_B, _N, 2)


# speedup vs baseline: 8.4422x; 1.2508x over previous
import jax
import jax.numpy as jnp
from jax.experimental import pallas as pl

_N = 17
_B = 256
_R = _N * _B


def _body(x_ref, sm_rows_ref, sm_cols_ref, sm_vals_ref,
          sp_rows_ref, sp_cols_ref, sp_vals_ref,
          w0_ref, b0_ref, w1_ref, b1_ref, w2_ref, b2_ref,
          w3_ref, b3_ref, w4_ref, b4_ref, w5_ref, b5_ref,
          out_ref):
    f32 = jnp.float32

    rr = jax.lax.broadcasted_iota(jnp.int32, (_R, _N), 0)
    nn = jax.lax.broadcasted_iota(jnp.int32, (_R, _N), 1)
    tile = (jnp.remainder(rr, _N) == nn).astype(f32)


    def coeff_cols(rows_ref, cols_ref, vals_ref):
        e = rows_ref.shape[1]
        ii = jax.lax.broadcasted_iota(jnp.int32, (_N, e), 0)
        rows = jnp.broadcast_to(rows_ref[...], (_N, e))
        cols = jnp.broadcast_to(cols_ref[...], (_N, e))
        vals = jnp.broadcast_to(vals_ref[...], (_N, e))
        on_row = rows == ii
        lo = jnp.sum(jnp.where(on_row & (cols == rows - 1), vals, 0.0),
                     axis=1, keepdims=True)
        di = jnp.sum(jnp.where(on_row & (cols == rows), vals, 0.0),
                     axis=1, keepdims=True)
        up = jnp.sum(jnp.where(on_row & (cols == rows + 1), vals, 0.0),
                     axis=1, keepdims=True)
        c = jnp.dot(tile, jnp.concatenate([lo, di, up], axis=1),
                    preferred_element_type=f32,
                    precision=jax.lax.Precision.HIGHEST)
        return c[:, 0:1], c[:, 1:2], c[:, 2:3]

    sm_lo, sm_di, sm_up = coeff_cols(sm_rows_ref, sm_cols_ref, sm_vals_ref)
    sp_lo, sp_di, sp_up = coeff_cols(sp_rows_ref, sp_cols_ref, sp_vals_ref)

    def layer(x, w_ref, b_ref, lo, di, up):
        y = jnp.dot(x.astype(jnp.bfloat16), w_ref[...].astype(jnp.bfloat16),
                    preferred_element_type=f32) + b_ref[...]
        d = y.shape[1]
        y_prev = jnp.concatenate([jnp.zeros((1, d), f32), y[:-1, :]], axis=0)
        y_next = jnp.concatenate([y[1:, :], jnp.zeros((1, d), f32)], axis=0)
        z = di * y + lo * y_prev + up * y_next
        return jnp.maximum(z, 0.0)

    x = x_ref[...]
    x = layer(x, w0_ref, b0_ref, sm_lo, sm_di, sm_up)
    x = layer(x, w1_ref, b1_ref, sm_lo, sm_di, sm_up)
    x = layer(x, w2_ref, b2_ref, sm_lo, sm_di, sm_up)
    x = layer(x, w3_ref, b3_ref, sp_lo, sp_di, sp_up)
    x = layer(x, w4_ref, b4_ref, sp_lo, sp_di, sp_up)
    x = layer(x, w5_ref, b5_ref, sp_lo, sp_di, sp_up)
    out_ref[...] = x


def kernel(H, sm_rows, sm_cols, sm_vals, sp_rows, sp_cols, sp_vals,
           W_enc0, b_enc0, W_enc1, b_enc1, W_enc2, b_enc2,
           W_dec0, b_dec0, W_dec1, b_dec1, W_dec2, b_dec2):
    f32 = jnp.float32
    x = H.reshape(_R, 2)
    coo = (sm_rows.reshape(1, -1), sm_cols.reshape(1, -1),
           sm_vals.reshape(1, -1), sp_rows.reshape(1, -1),
           sp_cols.reshape(1, -1), sp_vals.reshape(1, -1))
    wb = (W_enc0, b_enc0.reshape(1, -1), W_enc1, b_enc1.reshape(1, -1),
          W_enc2, b_enc2.reshape(1, -1), W_dec0, b_dec0.reshape(1, -1),
          W_dec1, b_dec1.reshape(1, -1), W_dec2, b_dec2.reshape(1, -1))

    out = pl.pallas_call(
        _body,
        out_shape=jax.ShapeDtypeStruct((_R, 2), f32),
    )(x, *coo, *wb)
    return out.reshape(_B, _N, 2)


# speedup vs baseline: 8.5117x; 1.0082x over previous
import jax
import jax.numpy as jnp
from jax.experimental import pallas as pl
from jax.experimental.pallas import tpu as pltpu

_N = 17
_B = 256
_R = _N * _B
_G = 2
_RB = _R // _G


def _body(x_ref, sm_rows_ref, sm_cols_ref, sm_vals_ref,
          sp_rows_ref, sp_cols_ref, sp_vals_ref,
          w0_ref, b0_ref, w1_ref, b1_ref, w2_ref, b2_ref,
          w3_ref, b3_ref, w4_ref, b4_ref, w5_ref, b5_ref,
          out_ref):
    f32 = jnp.float32

    rr = jax.lax.broadcasted_iota(jnp.int32, (_RB, _N), 0)
    nn = jax.lax.broadcasted_iota(jnp.int32, (_RB, _N), 1)
    tile = (jnp.remainder(rr, _N) == nn).astype(f32)


    def coeff_cols(rows_ref, cols_ref, vals_ref):
        e = rows_ref.shape[1]
        ii = jax.lax.broadcasted_iota(jnp.int32, (_N, e), 0)
        rows = jnp.broadcast_to(rows_ref[...], (_N, e))
        cols = jnp.broadcast_to(cols_ref[...], (_N, e))
        vals = jnp.broadcast_to(vals_ref[...], (_N, e))
        on_row = rows == ii
        lo = jnp.sum(jnp.where(on_row & (cols == rows - 1), vals, 0.0),
                     axis=1, keepdims=True)
        di = jnp.sum(jnp.where(on_row & (cols == rows), vals, 0.0),
                     axis=1, keepdims=True)
        up = jnp.sum(jnp.where(on_row & (cols == rows + 1), vals, 0.0),
                     axis=1, keepdims=True)
        c = jnp.dot(tile, jnp.concatenate([lo, di, up], axis=1),
                    preferred_element_type=f32,
                    precision=jax.lax.Precision.HIGHEST)
        return c[:, 0:1], c[:, 1:2], c[:, 2:3]

    sm_lo, sm_di, sm_up = coeff_cols(sm_rows_ref, sm_cols_ref, sm_vals_ref)
    sp_lo, sp_di, sp_up = coeff_cols(sp_rows_ref, sp_cols_ref, sp_vals_ref)

    def layer(x, w_ref, b_ref, lo, di, up):
        y = jnp.dot(x.astype(jnp.bfloat16), w_ref[...].astype(jnp.bfloat16),
                    preferred_element_type=f32) + b_ref[...]
        d = y.shape[1]
        y_prev = jnp.concatenate([jnp.zeros((1, d), f32), y[:-1, :]], axis=0)
        y_next = jnp.concatenate([y[1:, :], jnp.zeros((1, d), f32)], axis=0)
        z = di * y + lo * y_prev + up * y_next
        return jnp.maximum(z, 0.0)

    x = x_ref[...]
    x = layer(x, w0_ref, b0_ref, sm_lo, sm_di, sm_up)
    x = layer(x, w1_ref, b1_ref, sm_lo, sm_di, sm_up)
    x = layer(x, w2_ref, b2_ref, sm_lo, sm_di, sm_up)
    x = layer(x, w3_ref, b3_ref, sp_lo, sp_di, sp_up)
    x = layer(x, w4_ref, b4_ref, sp_lo, sp_di, sp_up)
    x = layer(x, w5_ref, b5_ref, sp_lo, sp_di, sp_up)
    out_ref[...] = x


def kernel(H, sm_rows, sm_cols, sm_vals, sp_rows, sp_cols, sp_vals,
           W_enc0, b_enc0, W_enc1, b_enc1, W_enc2, b_enc2,
           W_dec0, b_dec0, W_dec1, b_dec1, W_dec2, b_dec2):
    f32 = jnp.float32
    x = H.reshape(_R, 2)
    coo = (sm_rows.reshape(1, -1), sm_cols.reshape(1, -1),
           sm_vals.reshape(1, -1), sp_rows.reshape(1, -1),
           sp_cols.reshape(1, -1), sp_vals.reshape(1, -1))
    wb = (W_enc0, b_enc0.reshape(1, -1), W_enc1, b_enc1.reshape(1, -1),
          W_enc2, b_enc2.reshape(1, -1), W_dec0, b_dec0.reshape(1, -1),
          W_dec1, b_dec1.reshape(1, -1), W_dec2, b_dec2.reshape(1, -1))

    e = sm_rows.shape[0]
    full = lambda a: pl.BlockSpec(a.shape, lambda i: (0,) * a.ndim)
    in_specs = [pl.BlockSpec((_RB, 2), lambda i: (i, 0))]
    in_specs += [full(a) for a in coo]
    in_specs += [full(a) for a in wb]
    out = pl.pallas_call(
        _body,
        grid=(_G,),
        in_specs=in_specs,
        out_specs=pl.BlockSpec((_RB, 2), lambda i: (i, 0)),
        out_shape=jax.ShapeDtypeStruct((_R, 2), f32),
        compiler_params=pltpu.CompilerParams(
            dimension_semantics=("parallel",)),
    )(x, *coo, *wb)
    return out.reshape(_B, _N, 2)


# speedup vs baseline: 9.1901x; 1.0797x over previous
import jax
import jax.numpy as jnp
from jax.experimental import pallas as pl

_N = 17
_B = 256
_R = _N * _B


def _body(x_ref, sm_rows_ref, sm_cols_ref, sm_vals_ref,
          sp_rows_ref, sp_cols_ref, sp_vals_ref,
          w0_ref, b0_ref, w1_ref, b1_ref, w2_ref, b2_ref,
          w3_ref, b3_ref, w4_ref, b4_ref, w5_ref, b5_ref,
          out_ref):
    f32 = jnp.float32

    rr = jax.lax.broadcasted_iota(jnp.int32, (_R, _N), 0)
    nn = jax.lax.broadcasted_iota(jnp.int32, (_R, _N), 1)
    tile = (rr // _B == nn).astype(f32)

    def coeff_cols(rows_ref, cols_ref, vals_ref):
        e = rows_ref.shape[1]
        ii = jax.lax.broadcasted_iota(jnp.int32, (_N, e), 0)
        rows = jnp.broadcast_to(rows_ref[...], (_N, e))
        cols = jnp.broadcast_to(cols_ref[...], (_N, e))
        vals = jnp.broadcast_to(vals_ref[...], (_N, e))
        on_row = rows == ii
        lo = jnp.sum(jnp.where(on_row & (cols == rows - 1), vals, 0.0),
                     axis=1, keepdims=True)
        di = jnp.sum(jnp.where(on_row & (cols == rows), vals, 0.0),
                     axis=1, keepdims=True)
        up = jnp.sum(jnp.where(on_row & (cols == rows + 1), vals, 0.0),
                     axis=1, keepdims=True)
        c = jnp.dot(tile, jnp.concatenate([lo, di, up], axis=1),
                    preferred_element_type=f32,
                    precision=jax.lax.Precision.HIGHEST)
        return c[:, 0:1], c[:, 1:2], c[:, 2:3]

    sm_lo, sm_di, sm_up = coeff_cols(sm_rows_ref, sm_cols_ref, sm_vals_ref)
    sp_lo, sp_di, sp_up = coeff_cols(sp_rows_ref, sp_cols_ref, sp_vals_ref)

    def layer(x, w_ref, b_ref, lo, di, up):
        y = jnp.dot(x.astype(jnp.bfloat16), w_ref[...].astype(jnp.bfloat16),
                    preferred_element_type=f32) + b_ref[...]
        d = y.shape[1]
        y_prev = jnp.concatenate([jnp.zeros((_B, d), f32), y[:-_B, :]],
                                 axis=0)
        y_next = jnp.concatenate([y[_B:, :], jnp.zeros((_B, d), f32)],
                                 axis=0)
        z = di * y + lo * y_prev + up * y_next
        return jnp.maximum(z, 0.0)

    x = x_ref[...]
    x = layer(x, w0_ref, b0_ref, sm_lo, sm_di, sm_up)
    x = layer(x, w1_ref, b1_ref, sm_lo, sm_di, sm_up)
    x = layer(x, w2_ref, b2_ref, sm_lo, sm_di, sm_up)
    x = layer(x, w3_ref, b3_ref, sp_lo, sp_di, sp_up)
    x = layer(x, w4_ref, b4_ref, sp_lo, sp_di, sp_up)
    x = layer(x, w5_ref, b5_ref, sp_lo, sp_di, sp_up)
    out_ref[...] = x


def kernel(H, sm_rows, sm_cols, sm_vals, sp_rows, sp_cols, sp_vals,
           W_enc0, b_enc0, W_enc1, b_enc1, W_enc2, b_enc2,
           W_dec0, b_dec0, W_dec1, b_dec1, W_dec2, b_dec2):
    f32 = jnp.float32
    x = jnp.swapaxes(H, 0, 1).reshape(_R, 2)
    coo = (sm_rows.reshape(1, -1), sm_cols.reshape(1, -1),
           sm_vals.reshape(1, -1), sp_rows.reshape(1, -1),
           sp_cols.reshape(1, -1), sp_vals.reshape(1, -1))
    wb = (W_enc0, b_enc0.reshape(1, -1), W_enc1, b_enc1.reshape(1, -1),
          W_enc2, b_enc2.reshape(1, -1), W_dec0, b_dec0.reshape(1, -1),
          W_dec1, b_dec1.reshape(1, -1), W_dec2, b_dec2.reshape(1, -1))

    out = pl.pallas_call(
        _body,
        out_shape=jax.ShapeDtypeStruct((_R, 2), f32),
    )(x, *coo, *wb)
    return jnp.swapaxes(out.reshape(_N, _B, 2), 0, 1)
